# baseline (speedup 1.0000x reference)
import jax
import jax.numpy as jnp
from jax.experimental import pallas as pl
import jax.experimental.pallas.tpu as pltpu

N = 4096
NFEAT = 128
NHID = 32
NCLASS = 16
TILE = 512

_B1_R = N
_B2_R = N + 1
_W2_R0 = N + 8
_PK_ROWS = N + 40

_DEFAULT = jax.lax.Precision.DEFAULT


def _dot(a, b):
    return jax.lax.dot_general(a, b, (((1,), (0,)), ((), ())),
                               precision=_DEFAULT,
                               preferred_element_type=jnp.float32)


def _pack_kernel(x_ref, w1_ref, b1_ref, w2_ref, b2_ref, pk_ref):
    pk_ref[:N, :] = _dot(x_ref[...], w1_ref[...])
    pk_ref[pl.ds(_B1_R, 1), :] = b1_ref[...]
    z = jnp.zeros((1, NHID - NCLASS), jnp.float32)
    pk_ref[pl.ds(_B2_R, 1), :] = jnp.concatenate([b2_ref[...], z], axis=1)
    zw = jnp.zeros((NHID, NHID - NCLASS), jnp.float32)
    pk_ref[pl.ds(_W2_R0, NHID), :] = jnp.concatenate([w2_ref[...], zw], axis=1)
    pk_ref[pl.ds(_B2_R + 1, 6), :] = jnp.zeros((6, NHID), jnp.float32)


def _gcn_kernel(adj_ref, pk_ref, out_ref, sel_ref, h_ref):
    p = pl.program_id(0)
    t = pl.program_id(1)

    @pl.when((p == 0) & (t == 0))
    def _():
        sel_ref[...] = pk_ref[:N, :]

    @pl.when((p == 1) & (t == 0))
    def _():
        sel_ref[...] = _dot(h_ref[...], pk_ref[_W2_R0:_W2_R0 + NHID, :])

    acc = _dot(adj_ref[0], sel_ref[...])

    @pl.when(p == 0)
    def _():
        h_ref[pl.ds(t * TILE, TILE), :] = jnp.maximum(
            acc + pk_ref[_B1_R:_B1_R + 1, :], 0.0)

    @pl.when(p == 1)
    def _():
        o = acc[:, :NCLASS] + pk_ref[_B2_R:_B2_R + 1, :NCLASS]
        out_ref[...] = jnp.minimum(o, 0.0) - jnp.log1p(jnp.exp(-jnp.abs(o)))


@jax.jit
def kernel(x, adj_list, W1, b1, W2, b2):
    packed = pl.pallas_call(
        _pack_kernel,
        in_specs=[
            pl.BlockSpec((N, NFEAT), lambda: (0, 0)),
            pl.BlockSpec((NFEAT, NHID), lambda: (0, 0)),
            pl.BlockSpec((1, NHID), lambda: (0, 0)),
            pl.BlockSpec((NHID, NCLASS), lambda: (0, 0)),
            pl.BlockSpec((1, NCLASS), lambda: (0, 0)),
        ],
        out_specs=pl.BlockSpec((_PK_ROWS, NHID), lambda: (0, 0)),
        out_shape=jax.ShapeDtypeStruct((_PK_ROWS, NHID), jnp.float32),
    )(x, W1, b1.reshape(1, NHID), W2, b2.reshape(1, NCLASS))

    grid = (2, N // TILE)
    return pl.pallas_call(
        _gcn_kernel,
        grid=grid,
        in_specs=[
            pl.BlockSpec((1, TILE, N), lambda p, t: (p, t, 0)),
            pl.BlockSpec((_PK_ROWS, NHID), lambda p, t: (0, 0)),
        ],
        out_specs=pl.BlockSpec((TILE, NCLASS), lambda p, t: (p * t, 0)),
        out_shape=jax.ShapeDtypeStruct((N, NCLASS), jnp.float32),
        scratch_shapes=[
            pltpu.VMEM((N, NHID), jnp.float32),
            pltpu.VMEM((N, NHID), jnp.float32),
        ],
    )(adj_list, packed)
